# Initial kernel scaffold; baseline (speedup 1.0000x reference)
import jax
import jax.numpy as jnp
from jax.experimental import pallas as pl


def kernel(Y, X, src_x, src_y):
    raise NotImplementedError("write your pallas kernel here")


# speedup vs baseline: 2.7945x; 2.7945x over previous
import jax
import jax.numpy as jnp
from jax import lax
from jax.experimental import pallas as pl
from jax.experimental.pallas import tpu as pltpu

_R = 256
_NSRC = 32


def _body(src_x_ref, src_y_ref, x_ref, y_ref, out_ref):
    b = pl.program_id(0)
    rb = pl.program_id(1)
    r0 = rb * _R
    out_ref[...] = y_ref[...]
    col = lax.broadcasted_iota(jnp.int32, (1, 2048), 1)
    for i in range(_NSRC):
        sx = src_x_ref[i]
        sy = src_y_ref[i]

        @pl.when(jnp.logical_and(sx >= r0, sx < r0 + _R))
        def _():
            xl = sx - r0
            xv = x_ref[b, i]
            row = out_ref[0, pl.ds(xl, 1), :]
            out_ref[0, pl.ds(xl, 1), :] = row + jnp.where(col == sy, xv, 0.0)


def kernel(Y, X, src_x, src_y):
    B, H, W = Y.shape
    grid = (B, H // _R)
    return pl.pallas_call(
        _body,
        grid=grid,
        in_specs=[
            pl.BlockSpec(memory_space=pltpu.SMEM),
            pl.BlockSpec(memory_space=pltpu.SMEM),
            pl.BlockSpec(memory_space=pltpu.SMEM),
            pl.BlockSpec((1, _R, W), lambda b, r: (b, r, 0)),
        ],
        out_specs=pl.BlockSpec((1, _R, W), lambda b, r: (b, r, 0)),
        out_shape=jax.ShapeDtypeStruct(Y.shape, Y.dtype),
        compiler_params=pltpu.CompilerParams(
            dimension_semantics=("parallel", "parallel"),
        ),
    )(src_x, src_y, X, Y)


# speedup vs baseline: 3.2637x; 1.1679x over previous
import jax
import jax.numpy as jnp
from jax import lax
from jax.experimental import pallas as pl
from jax.experimental.pallas import tpu as pltpu

_R = 512
_NSRC = 32


def _body(src_x_ref, src_y_ref, x_ref, y_ref, out_ref):
    b = pl.program_id(0)
    rb = pl.program_id(1)
    r0 = rb * _R
    out_ref[...] = y_ref[...]
    col = lax.broadcasted_iota(jnp.int32, (1, 2048), 1)
    for i in range(_NSRC):
        sx = src_x_ref[i]
        sy = src_y_ref[i]

        @pl.when(jnp.logical_and(sx >= r0, sx < r0 + _R))
        def _():
            xl = sx - r0
            xv = x_ref[b, i]
            row = out_ref[0, pl.ds(xl, 1), :]
            out_ref[0, pl.ds(xl, 1), :] = row + jnp.where(col == sy, xv, 0.0)


def kernel(Y, X, src_x, src_y):
    B, H, W = Y.shape
    grid = (B, H // _R)
    return pl.pallas_call(
        _body,
        grid=grid,
        in_specs=[
            pl.BlockSpec(memory_space=pltpu.SMEM),
            pl.BlockSpec(memory_space=pltpu.SMEM),
            pl.BlockSpec(memory_space=pltpu.SMEM),
            pl.BlockSpec((1, _R, W), lambda b, r: (b, r, 0)),
        ],
        out_specs=pl.BlockSpec((1, _R, W), lambda b, r: (b, r, 0)),
        out_shape=jax.ShapeDtypeStruct(Y.shape, Y.dtype),
        compiler_params=pltpu.CompilerParams(
            dimension_semantics=("parallel", "parallel"),
        ),
    )(src_x, src_y, X, Y)


# speedup vs baseline: 3.3843x; 1.0370x over previous
import jax
import jax.numpy as jnp
from jax import lax
from jax.experimental import pallas as pl
from jax.experimental.pallas import tpu as pltpu

_R = 1024
_NSRC = 32


def _body(src_x_ref, src_y_ref, x_ref, y_ref, out_ref):
    b = pl.program_id(0)
    rb = pl.program_id(1)
    r0 = rb * _R
    out_ref[...] = y_ref[...]
    col = lax.broadcasted_iota(jnp.int32, (1, 2048), 1)
    for i in range(_NSRC):
        sx = src_x_ref[i]
        sy = src_y_ref[i]

        @pl.when(jnp.logical_and(sx >= r0, sx < r0 + _R))
        def _():
            xl = sx - r0
            xv = x_ref[b, i]
            row = out_ref[0, pl.ds(xl, 1), :]
            out_ref[0, pl.ds(xl, 1), :] = row + jnp.where(col == sy, xv, 0.0)


def kernel(Y, X, src_x, src_y):
    B, H, W = Y.shape
    grid = (B, H // _R)
    return pl.pallas_call(
        _body,
        grid=grid,
        in_specs=[
            pl.BlockSpec(memory_space=pltpu.SMEM),
            pl.BlockSpec(memory_space=pltpu.SMEM),
            pl.BlockSpec(memory_space=pltpu.SMEM),
            pl.BlockSpec((1, _R, W), lambda b, r: (b, r, 0)),
        ],
        out_specs=pl.BlockSpec((1, _R, W), lambda b, r: (b, r, 0)),
        out_shape=jax.ShapeDtypeStruct(Y.shape, Y.dtype),
        compiler_params=pltpu.CompilerParams(
            dimension_semantics=("parallel", "parallel"),
        ),
    )(src_x, src_y, X, Y)
